# Initial kernel scaffold; baseline (speedup 1.0000x reference)
import jax
import jax.numpy as jnp
from jax.experimental import pallas as pl


def kernel(x, kohonen_weights, grossberg_weights):
    raise NotImplementedError("write your pallas kernel here")


# speedup vs baseline: 1.0248x; 1.0248x over previous
import functools

import jax
import jax.numpy as jnp
from jax import lax
from jax.experimental import pallas as pl
from jax.experimental.pallas import tpu as pltpu, tpu_sc as plsc

B = 16384
IN = 32
HIDDEN = 8192
OUT = 64

BB = 256
HALF = HIDDEN // 2
CH = 1024
NCH_HALF = HALF // CH


def _bf16_rne(v):
    bits = lax.bitcast_convert_type(v, jnp.uint32)
    r = (bits + jnp.uint32(0x7FFF) + ((bits >> 16) & jnp.uint32(1))) \
        & jnp.uint32(0xFFFF0000)
    return lax.bitcast_convert_type(r, jnp.float32)


def _half_argmin(xb, kt_ref, x2, w2_ref, base):
    run_min = jnp.full((BB, 1), jnp.inf, dtype=jnp.float32)
    run_idx = jnp.zeros((BB, 1), dtype=jnp.int32)
    for c in range(NCH_HALF):
        lo = base + c * CH
        kc = kt_ref[:, lo:lo + CH]
        m = jnp.dot(xb, kc.astype(jnp.bfloat16),
                    preferred_element_type=jnp.float32)
        w2 = w2_ref[0, lo:lo + CH].reshape(1, CH)
        t = x2 + w2
        d2 = jnp.maximum(t - 2.0 * m, 0.0)
        dist = jnp.sqrt(d2)
        cmin = jnp.min(dist, axis=1, keepdims=True)
        ids = lax.broadcasted_iota(jnp.int32, dist.shape, 1) + lo
        cidx = jnp.min(jnp.where(dist == cmin, ids, HIDDEN), axis=1,
                       keepdims=True)
        upd = cmin < run_min
        run_idx = jnp.where(upd, cidx, run_idx)
        run_min = jnp.where(upd, cmin, run_min)
    return run_min, run_idx


def _winner_body(x_ref, kt_ref, x2_ref, w2_ref, win_ref):
    xb = x_ref[...].astype(jnp.bfloat16)
    x2 = x2_ref[0].reshape(BB, 1)
    m1, i1 = _half_argmin(xb, kt_ref, x2, w2_ref, 0)
    m2, i2 = _half_argmin(xb, kt_ref, x2, w2_ref, HALF)
    r0 = _bf16_rne(m1)
    win = jnp.where(m2 < r0, i2, i1)
    win_ref[...] = win.reshape(1, 1, BB)


_winner_call = pl.pallas_call(
    _winner_body,
    grid=(B // BB,),
    in_specs=[
        pl.BlockSpec((BB, IN), lambda i: (i, 0)),
        pl.BlockSpec((IN, HIDDEN), lambda i: (0, 0)),
        pl.BlockSpec((1, BB), lambda i: (0, i)),
        pl.BlockSpec((1, HIDDEN), lambda i: (0, 0)),
    ],
    out_specs=pl.BlockSpec((1, 1, BB), lambda i: (i, 0, 0)),
    out_shape=jax.ShapeDtypeStruct((B // BB, 1, BB), jnp.int32),
)


_info = plsc.get_sparse_core_info()
_NC, _NS = _info.num_cores, _info.num_subcores
_NW = _NC * _NS
_BPW = B // _NW
_DPAD = 128
_CHI = 128
_NCHI = _BPW // _CHI

_mesh = plsc.VectorSubcoreMesh(core_axis_name="c", subcore_axis_name="s")


@functools.partial(
    pl.kernel,
    mesh=_mesh,
    out_type=jax.ShapeDtypeStruct((B, _DPAD), jnp.float32),
    scratch_types=[
        pltpu.VMEM((_NCHI, _CHI), jnp.int32),
        pltpu.VMEM((_BPW, _DPAD), jnp.float32),
        pltpu.SemaphoreType.DMA,
    ],
)
def _gather_call(table_hbm, idx_hbm, out_hbm, idx_v, rows_v, sem):
    wid = lax.axis_index("s") * _NC + lax.axis_index("c")
    pltpu.sync_copy(idx_hbm.at[wid], idx_v)
    handles = [
        pltpu.async_copy(
            table_hbm.at[idx_v.at[j]],
            rows_v.at[pl.ds(j * _CHI, _CHI)],
            sem,
        )
        for j in range(_NCHI)
    ]
    for h in handles:
        h.wait()
    pltpu.sync_copy(rows_v, out_hbm.at[pl.ds(wid * _BPW, _BPW)])


def kernel(x, kohonen_weights, grossberg_weights):
    x2 = jnp.sum(x * x, axis=1)[None, :]
    w2 = jnp.sum(kohonen_weights * kohonen_weights, axis=1)[None, :]
    winners = _winner_call(x, kohonen_weights.T, x2, w2).reshape(B)
    table = jnp.pad(grossberg_weights.T, ((0, 0), (0, _DPAD - OUT)))
    output = _gather_call(table, winners.reshape(_NW, _NCHI, _CHI))[:, :OUT]
    return (output, winners)


# speedup vs baseline: 1.0625x; 1.0369x over previous
import functools

import jax
import jax.numpy as jnp
from jax import lax
from jax.experimental import pallas as pl
from jax.experimental.pallas import tpu as pltpu, tpu_sc as plsc

B = 16384
IN = 32
HIDDEN = 8192
OUT = 64

BB = 256
HALF = HIDDEN // 2
CH = 1024
NCH_HALF = HALF // CH


def _bf16_rne(v):
    bits = lax.bitcast_convert_type(v, jnp.uint32)
    r = (bits + jnp.uint32(0x7FFF) + ((bits >> 16) & jnp.uint32(1))) \
        & jnp.uint32(0xFFFF0000)
    return lax.bitcast_convert_type(r, jnp.float32)


def _half_argmin(xb, kt2_ref, x2, w2_ref, base):
    run_min = jnp.full((BB, 1), jnp.inf, dtype=jnp.float32)
    run_idx = jnp.zeros((BB, 1), dtype=jnp.int32)
    for c in range(NCH_HALF):
        lo = base + c * CH
        kc = kt2_ref[:, lo:lo + CH]
        m2x = jnp.dot(xb, kc, preferred_element_type=jnp.float32)
        w2 = w2_ref[0, lo:lo + CH].reshape(1, CH)
        t = x2 + w2
        d2 = jnp.maximum(t - m2x, 0.0)
        dist = jnp.sqrt(d2)
        cmin = jnp.min(dist, axis=1, keepdims=True)
        ids = lax.broadcasted_iota(jnp.int32, dist.shape, 1)
        cidx = jnp.min(jnp.where(dist == cmin, ids, HIDDEN), axis=1,
                       keepdims=True) + lo
        upd = cmin < run_min
        run_idx = jnp.where(upd, cidx, run_idx)
        run_min = jnp.where(upd, cmin, run_min)
    return run_min, run_idx


def _winner_body(x_ref, kt2_ref, x2_ref, w2_ref, win_ref):
    xb = x_ref[...].astype(jnp.bfloat16)
    x2 = x2_ref[0].reshape(BB, 1)
    m1, i1 = _half_argmin(xb, kt2_ref, x2, w2_ref, 0)
    m2, i2 = _half_argmin(xb, kt2_ref, x2, w2_ref, HALF)
    r0 = _bf16_rne(m1)
    win = jnp.where(m2 < r0, i2, i1)
    win_ref[...] = win.reshape(1, 1, BB)


_winner_call = pl.pallas_call(
    _winner_body,
    grid=(B // BB,),
    in_specs=[
        pl.BlockSpec((BB, IN), lambda i: (i, 0)),
        pl.BlockSpec((IN, HIDDEN), lambda i: (0, 0)),
        pl.BlockSpec((1, BB), lambda i: (0, i)),
        pl.BlockSpec((1, HIDDEN), lambda i: (0, 0)),
    ],
    out_specs=pl.BlockSpec((1, 1, BB), lambda i: (i, 0, 0)),
    out_shape=jax.ShapeDtypeStruct((B // BB, 1, BB), jnp.int32),
)


_info = plsc.get_sparse_core_info()
_NC, _NS = _info.num_cores, _info.num_subcores
_NW = _NC * _NS
_BPW = B // _NW
_DPAD = 128
_CHI = 128
_NCHI = _BPW // _CHI

_mesh = plsc.VectorSubcoreMesh(core_axis_name="c", subcore_axis_name="s")


@functools.partial(
    pl.kernel,
    mesh=_mesh,
    out_type=jax.ShapeDtypeStruct((B, _DPAD), jnp.float32),
    scratch_types=[
        pltpu.VMEM((_NCHI, _CHI), jnp.int32),
        pltpu.VMEM((_BPW, _DPAD), jnp.float32),
        pltpu.SemaphoreType.DMA,
    ],
)
def _gather_call(table_hbm, idx_hbm, out_hbm, idx_v, rows_v, sem):
    wid = lax.axis_index("s") * _NC + lax.axis_index("c")
    pltpu.sync_copy(idx_hbm.at[wid], idx_v)
    handles = [
        pltpu.async_copy(
            table_hbm.at[idx_v.at[j]],
            rows_v.at[pl.ds(j * _CHI, _CHI)],
            sem,
        )
        for j in range(_NCHI)
    ]
    for h in handles:
        h.wait()
    pltpu.sync_copy(rows_v, out_hbm.at[pl.ds(wid * _BPW, _BPW)])


def kernel(x, kohonen_weights, grossberg_weights):
    x2 = jnp.sum(x * x, axis=1)[None, :]
    w2 = jnp.sum(kohonen_weights * kohonen_weights, axis=1)[None, :]
    kt2 = kohonen_weights.T.astype(jnp.bfloat16) * jnp.bfloat16(2.0)
    winners = _winner_call(x, kt2, x2, w2).reshape(B)
    table = jnp.pad(grossberg_weights.T, ((0, 0), (0, _DPAD - OUT)))
    output = _gather_call(table, winners.reshape(_NW, _NCHI, _CHI))[:, :OUT]
    return (output, winners)


# speedup vs baseline: 1.1265x; 1.0603x over previous
import functools

import jax
import jax.numpy as jnp
from jax import lax
from jax.experimental import pallas as pl
from jax.experimental.pallas import tpu as pltpu, tpu_sc as plsc

B = 16384
IN = 32
HIDDEN = 8192
OUT = 64

BB = 128
HALF = HIDDEN // 2
CH = 128
NCH_HALF = HALF // CH


def _bf16_rne(v):
    bits = lax.bitcast_convert_type(v, jnp.uint32)
    r = (bits + jnp.uint32(0x7FFF) + ((bits >> 16) & jnp.uint32(1))) \
        & jnp.uint32(0xFFFF0000)
    return lax.bitcast_convert_type(r, jnp.float32)


def _half_argmin(xb, kt2_ref, x2, w2_ref, base):
    acc_v = jnp.full((BB, CH), jnp.inf, dtype=jnp.float32)
    acc_c = jnp.zeros((BB, CH), dtype=jnp.int32)
    for c in range(NCH_HALF):
        lo = base + c * CH
        kc = kt2_ref[:, lo:lo + CH]
        m2x = jnp.dot(xb, kc, preferred_element_type=jnp.float32)
        w2 = w2_ref[0, lo:lo + CH].reshape(1, CH)
        t = x2 + w2
        dist = jnp.sqrt(jnp.maximum(t - m2x, 0.0))
        upd = dist < acc_v
        acc_v = jnp.where(upd, dist, acc_v)
        acc_c = jnp.where(upd, c, acc_c)
    gid = acc_c * CH + lax.broadcasted_iota(jnp.int32, (BB, CH), 1) + base
    run_min = jnp.min(acc_v, axis=1, keepdims=True)
    run_idx = jnp.min(jnp.where(acc_v == run_min, gid, HIDDEN), axis=1,
                      keepdims=True)
    return run_min, run_idx


def _winner_body(x_ref, kt2_ref, x2_ref, w2_ref, win_ref):
    xb = x_ref[...].astype(jnp.bfloat16)
    x2 = x2_ref[0].reshape(BB, 1)
    m1, i1 = _half_argmin(xb, kt2_ref, x2, w2_ref, 0)
    m2, i2 = _half_argmin(xb, kt2_ref, x2, w2_ref, HALF)
    r0 = _bf16_rne(m1)
    win = jnp.where(m2 < r0, i2, i1)
    win_ref[...] = win.reshape(1, 1, BB)


_winner_call = pl.pallas_call(
    _winner_body,
    grid=(B // BB,),
    in_specs=[
        pl.BlockSpec((BB, IN), lambda i: (i, 0)),
        pl.BlockSpec((IN, HIDDEN), lambda i: (0, 0)),
        pl.BlockSpec((1, BB), lambda i: (0, i)),
        pl.BlockSpec((1, HIDDEN), lambda i: (0, 0)),
    ],
    out_specs=pl.BlockSpec((1, 1, BB), lambda i: (i, 0, 0)),
    out_shape=jax.ShapeDtypeStruct((B // BB, 1, BB), jnp.int32),
)


_info = plsc.get_sparse_core_info()
_NC, _NS = _info.num_cores, _info.num_subcores
_NW = _NC * _NS
_BPW = B // _NW
_DPAD = 128
_CHI = 128
_NCHI = _BPW // _CHI

_mesh = plsc.VectorSubcoreMesh(core_axis_name="c", subcore_axis_name="s")


@functools.partial(
    pl.kernel,
    mesh=_mesh,
    out_type=jax.ShapeDtypeStruct((B, _DPAD), jnp.float32),
    scratch_types=[
        pltpu.VMEM((_NCHI, _CHI), jnp.int32),
        pltpu.VMEM((_BPW, _DPAD), jnp.float32),
        pltpu.SemaphoreType.DMA,
    ],
)
def _gather_call(table_hbm, idx_hbm, out_hbm, idx_v, rows_v, sem):
    wid = lax.axis_index("s") * _NC + lax.axis_index("c")
    pltpu.sync_copy(idx_hbm.at[wid], idx_v)
    handles = [
        pltpu.async_copy(
            table_hbm.at[idx_v.at[j]],
            rows_v.at[pl.ds(j * _CHI, _CHI)],
            sem,
        )
        for j in range(_NCHI)
    ]
    for h in handles:
        h.wait()
    pltpu.sync_copy(rows_v, out_hbm.at[pl.ds(wid * _BPW, _BPW)])


def kernel(x, kohonen_weights, grossberg_weights):
    x2 = jnp.sum(x * x, axis=1)[None, :]
    w2 = jnp.sum(kohonen_weights * kohonen_weights, axis=1)[None, :]
    kt2 = kohonen_weights.T.astype(jnp.bfloat16) * jnp.bfloat16(2.0)
    winners = _winner_call(x, kt2, x2, w2).reshape(B)
    table = jnp.pad(grossberg_weights.T, ((0, 0), (0, _DPAD - OUT)))
    output = _gather_call(table, winners.reshape(_NW, _NCHI, _CHI))[:, :OUT]
    return (output, winners)


# speedup vs baseline: 1.1418x; 1.0135x over previous
import functools

import jax
import jax.numpy as jnp
from jax import lax
from jax.experimental import pallas as pl
from jax.experimental.pallas import tpu as pltpu, tpu_sc as plsc

B = 16384
IN = 32
HIDDEN = 8192
OUT = 64

BB = 256
HALF = HIDDEN // 2
CH = 128
NCH_HALF = HALF // CH


def _bf16_rne(v):
    bits = lax.bitcast_convert_type(v, jnp.uint32)
    r = (bits + jnp.uint32(0x7FFF) + ((bits >> 16) & jnp.uint32(1))) \
        & jnp.uint32(0xFFFF0000)
    return lax.bitcast_convert_type(r, jnp.float32)


def _half_argmin(xb, kt2_ref, x2, w2_ref, base):
    acc_v = jnp.full((BB, CH), jnp.inf, dtype=jnp.float32)
    acc_c = jnp.zeros((BB, CH), dtype=jnp.int32)
    for c in range(NCH_HALF):
        lo = base + c * CH
        kc = kt2_ref[:, lo:lo + CH]
        m2x = jnp.dot(xb, kc, preferred_element_type=jnp.float32)
        w2 = w2_ref[0, lo:lo + CH].reshape(1, CH)
        t = x2 + w2
        dist = jnp.sqrt(jnp.maximum(t - m2x, 0.0))
        upd = dist < acc_v
        acc_v = jnp.where(upd, dist, acc_v)
        acc_c = jnp.where(upd, c, acc_c)
    gid = acc_c * CH + lax.broadcasted_iota(jnp.int32, (BB, CH), 1) + base
    run_min = jnp.min(acc_v, axis=1, keepdims=True)
    run_idx = jnp.min(jnp.where(acc_v == run_min, gid, HIDDEN), axis=1,
                      keepdims=True)
    return run_min, run_idx


def _winner_body(x_ref, kt2_ref, x2_ref, w2_ref, win_ref):
    xb = x_ref[...].astype(jnp.bfloat16)
    x2 = x2_ref[0].reshape(BB, 1)
    m1, i1 = _half_argmin(xb, kt2_ref, x2, w2_ref, 0)
    m2, i2 = _half_argmin(xb, kt2_ref, x2, w2_ref, HALF)
    r0 = _bf16_rne(m1)
    win = jnp.where(m2 < r0, i2, i1)
    win_ref[...] = win.reshape(1, 1, BB)


_winner_call = pl.pallas_call(
    _winner_body,
    grid=(B // BB,),
    in_specs=[
        pl.BlockSpec((BB, IN), lambda i: (i, 0)),
        pl.BlockSpec((IN, HIDDEN), lambda i: (0, 0)),
        pl.BlockSpec((1, BB), lambda i: (0, i)),
        pl.BlockSpec((1, HIDDEN), lambda i: (0, 0)),
    ],
    out_specs=pl.BlockSpec((1, 1, BB), lambda i: (i, 0, 0)),
    out_shape=jax.ShapeDtypeStruct((B // BB, 1, BB), jnp.int32),
)


_info = plsc.get_sparse_core_info()
_NC, _NS = _info.num_cores, _info.num_subcores
_NW = _NC * _NS
_BPW = B // _NW
_DPAD = 128
_CHI = 128
_NCHI = _BPW // _CHI

_mesh = plsc.VectorSubcoreMesh(core_axis_name="c", subcore_axis_name="s")


@functools.partial(
    pl.kernel,
    mesh=_mesh,
    out_type=jax.ShapeDtypeStruct((B, _DPAD), jnp.float32),
    scratch_types=[
        pltpu.VMEM((_NCHI, _CHI), jnp.int32),
        pltpu.VMEM((_BPW, _DPAD), jnp.float32),
        pltpu.SemaphoreType.DMA,
    ],
)
def _gather_call(table_hbm, idx_hbm, out_hbm, idx_v, rows_v, sem):
    wid = lax.axis_index("s") * _NC + lax.axis_index("c")
    pltpu.sync_copy(idx_hbm.at[wid], idx_v)
    handles = [
        pltpu.async_copy(
            table_hbm.at[idx_v.at[j]],
            rows_v.at[pl.ds(j * _CHI, _CHI)],
            sem,
        )
        for j in range(_NCHI)
    ]
    for h in handles:
        h.wait()
    pltpu.sync_copy(rows_v, out_hbm.at[pl.ds(wid * _BPW, _BPW)])


def kernel(x, kohonen_weights, grossberg_weights):
    x2 = jnp.sum(x * x, axis=1)[None, :]
    w2 = jnp.sum(kohonen_weights * kohonen_weights, axis=1)[None, :]
    kt2 = kohonen_weights.T.astype(jnp.bfloat16) * jnp.bfloat16(2.0)
    winners = _winner_call(x, kt2, x2, w2).reshape(B)
    table = jnp.pad(grossberg_weights.T, ((0, 0), (0, _DPAD - OUT)))
    output = _gather_call(table, winners.reshape(_NW, _NCHI, _CHI))[:, :OUT]
    return (output, winners)


# speedup vs baseline: 1.2186x; 1.0673x over previous
import functools

import jax
import jax.numpy as jnp
from jax import lax
from jax.experimental import pallas as pl
from jax.experimental.pallas import tpu as pltpu, tpu_sc as plsc

B = 16384
IN = 32
HIDDEN = 8192
OUT = 64

BB = 512
HALF = HIDDEN // 2
CH = 128
NCH_HALF = HALF // CH


def _bf16_rne(v):
    bits = lax.bitcast_convert_type(v, jnp.uint32)
    r = (bits + jnp.uint32(0x7FFF) + ((bits >> 16) & jnp.uint32(1))) \
        & jnp.uint32(0xFFFF0000)
    return lax.bitcast_convert_type(r, jnp.float32)


def _half_argmin(xb, kt2_ref, x2, w2_ref, base):
    acc_v = jnp.full((BB, CH), jnp.inf, dtype=jnp.float32)
    acc_c = jnp.zeros((BB, CH), dtype=jnp.int32)
    for c in range(NCH_HALF):
        lo = base + c * CH
        kc = kt2_ref[:, lo:lo + CH]
        m2x = jnp.dot(xb, kc, preferred_element_type=jnp.float32)
        w2 = w2_ref[0, lo:lo + CH].reshape(1, CH)
        t = x2 + w2
        dist = jnp.sqrt(jnp.maximum(t - m2x, 0.0))
        upd = dist < acc_v
        acc_v = jnp.where(upd, dist, acc_v)
        acc_c = jnp.where(upd, c, acc_c)
    gid = acc_c * CH + lax.broadcasted_iota(jnp.int32, (BB, CH), 1) + base
    run_min = jnp.min(acc_v, axis=1, keepdims=True)
    run_idx = jnp.min(jnp.where(acc_v == run_min, gid, HIDDEN), axis=1,
                      keepdims=True)
    return run_min, run_idx


def _winner_body(x_ref, kt2_ref, x2_ref, w2_ref, win_ref):
    xb = x_ref[...].astype(jnp.bfloat16)
    x2 = x2_ref[0].reshape(BB, 1)
    m1, i1 = _half_argmin(xb, kt2_ref, x2, w2_ref, 0)
    m2, i2 = _half_argmin(xb, kt2_ref, x2, w2_ref, HALF)
    r0 = _bf16_rne(m1)
    win = jnp.where(m2 < r0, i2, i1)
    win_ref[...] = win.reshape(1, 1, BB)


_winner_call = pl.pallas_call(
    _winner_body,
    grid=(B // BB,),
    in_specs=[
        pl.BlockSpec((BB, IN), lambda i: (i, 0)),
        pl.BlockSpec((IN, HIDDEN), lambda i: (0, 0)),
        pl.BlockSpec((1, BB), lambda i: (0, i)),
        pl.BlockSpec((1, HIDDEN), lambda i: (0, 0)),
    ],
    out_specs=pl.BlockSpec((1, 1, BB), lambda i: (i, 0, 0)),
    out_shape=jax.ShapeDtypeStruct((B // BB, 1, BB), jnp.int32),
)


_info = plsc.get_sparse_core_info()
_NC, _NS = _info.num_cores, _info.num_subcores
_NW = _NC * _NS
_BPW = B // _NW
_DPAD = 128
_CHI = 128
_NCHI = _BPW // _CHI

_mesh = plsc.VectorSubcoreMesh(core_axis_name="c", subcore_axis_name="s")


@functools.partial(
    pl.kernel,
    mesh=_mesh,
    out_type=jax.ShapeDtypeStruct((B, _DPAD), jnp.float32),
    scratch_types=[
        pltpu.VMEM((_NCHI, _CHI), jnp.int32),
        pltpu.VMEM((_BPW, _DPAD), jnp.float32),
        pltpu.SemaphoreType.DMA,
    ],
)
def _gather_call(table_hbm, idx_hbm, out_hbm, idx_v, rows_v, sem):
    wid = lax.axis_index("s") * _NC + lax.axis_index("c")
    pltpu.sync_copy(idx_hbm.at[wid], idx_v)
    handles = [
        pltpu.async_copy(
            table_hbm.at[idx_v.at[j]],
            rows_v.at[pl.ds(j * _CHI, _CHI)],
            sem,
        )
        for j in range(_NCHI)
    ]
    for h in handles:
        h.wait()
    pltpu.sync_copy(rows_v, out_hbm.at[pl.ds(wid * _BPW, _BPW)])


def kernel(x, kohonen_weights, grossberg_weights):
    x2 = jnp.sum(x * x, axis=1)[None, :]
    w2 = jnp.sum(kohonen_weights * kohonen_weights, axis=1)[None, :]
    kt2 = kohonen_weights.T.astype(jnp.bfloat16) * jnp.bfloat16(2.0)
    winners = _winner_call(x, kt2, x2, w2).reshape(B)
    table = jnp.pad(grossberg_weights.T, ((0, 0), (0, _DPAD - OUT)))
    output = _gather_call(table, winners.reshape(_NW, _NCHI, _CHI))[:, :OUT]
    return (output, winners)


# speedup vs baseline: 1.2519x; 1.0273x over previous
import functools

import jax
import jax.numpy as jnp
from jax import lax
from jax.experimental import pallas as pl
from jax.experimental.pallas import tpu as pltpu, tpu_sc as plsc

B = 16384
IN = 32
HIDDEN = 8192
OUT = 64

BB = 1024
HALF = HIDDEN // 2
CH = 128
NCH_HALF = HALF // CH


def _bf16_rne(v):
    bits = lax.bitcast_convert_type(v, jnp.uint32)
    r = (bits + jnp.uint32(0x7FFF) + ((bits >> 16) & jnp.uint32(1))) \
        & jnp.uint32(0xFFFF0000)
    return lax.bitcast_convert_type(r, jnp.float32)


def _half_argmin(xb, kt2_ref, x2, w2_ref, base):
    acc_v = jnp.full((BB, CH), jnp.inf, dtype=jnp.float32)
    acc_c = jnp.zeros((BB, CH), dtype=jnp.int32)
    for c in range(NCH_HALF):
        lo = base + c * CH
        kc = kt2_ref[:, lo:lo + CH]
        m2x = jnp.dot(xb, kc, preferred_element_type=jnp.float32)
        w2 = w2_ref[0, lo:lo + CH].reshape(1, CH)
        t = x2 + w2
        dist = jnp.sqrt(jnp.maximum(t - m2x, 0.0))
        upd = dist < acc_v
        acc_v = jnp.where(upd, dist, acc_v)
        acc_c = jnp.where(upd, c, acc_c)
    gid = acc_c * CH + lax.broadcasted_iota(jnp.int32, (BB, CH), 1) + base
    run_min = jnp.min(acc_v, axis=1, keepdims=True)
    run_idx = jnp.min(jnp.where(acc_v == run_min, gid, HIDDEN), axis=1,
                      keepdims=True)
    return run_min, run_idx


def _winner_body(x_ref, kt2_ref, x2_ref, w2_ref, win_ref):
    xb = x_ref[...].astype(jnp.bfloat16)
    x2 = x2_ref[0].reshape(BB, 1)
    m1, i1 = _half_argmin(xb, kt2_ref, x2, w2_ref, 0)
    m2, i2 = _half_argmin(xb, kt2_ref, x2, w2_ref, HALF)
    r0 = _bf16_rne(m1)
    win = jnp.where(m2 < r0, i2, i1)
    win_ref[...] = win.reshape(1, 1, BB)


_winner_call = pl.pallas_call(
    _winner_body,
    grid=(B // BB,),
    in_specs=[
        pl.BlockSpec((BB, IN), lambda i: (i, 0)),
        pl.BlockSpec((IN, HIDDEN), lambda i: (0, 0)),
        pl.BlockSpec((1, BB), lambda i: (0, i)),
        pl.BlockSpec((1, HIDDEN), lambda i: (0, 0)),
    ],
    out_specs=pl.BlockSpec((1, 1, BB), lambda i: (i, 0, 0)),
    out_shape=jax.ShapeDtypeStruct((B // BB, 1, BB), jnp.int32),
)


_info = plsc.get_sparse_core_info()
_NC, _NS = _info.num_cores, _info.num_subcores
_NW = _NC * _NS
_BPW = B // _NW
_DPAD = 128
_CHI = 128
_NCHI = _BPW // _CHI

_mesh = plsc.VectorSubcoreMesh(core_axis_name="c", subcore_axis_name="s")


@functools.partial(
    pl.kernel,
    mesh=_mesh,
    out_type=jax.ShapeDtypeStruct((B, _DPAD), jnp.float32),
    scratch_types=[
        pltpu.VMEM((_NCHI, _CHI), jnp.int32),
        pltpu.VMEM((_BPW, _DPAD), jnp.float32),
        pltpu.SemaphoreType.DMA,
    ],
)
def _gather_call(table_hbm, idx_hbm, out_hbm, idx_v, rows_v, sem):
    wid = lax.axis_index("s") * _NC + lax.axis_index("c")
    pltpu.sync_copy(idx_hbm.at[wid], idx_v)
    handles = [
        pltpu.async_copy(
            table_hbm.at[idx_v.at[j]],
            rows_v.at[pl.ds(j * _CHI, _CHI)],
            sem,
        )
        for j in range(_NCHI)
    ]
    for h in handles:
        h.wait()
    pltpu.sync_copy(rows_v, out_hbm.at[pl.ds(wid * _BPW, _BPW)])


def kernel(x, kohonen_weights, grossberg_weights):
    x2 = jnp.sum(x * x, axis=1)[None, :]
    w2 = jnp.sum(kohonen_weights * kohonen_weights, axis=1)[None, :]
    kt2 = kohonen_weights.T.astype(jnp.bfloat16) * jnp.bfloat16(2.0)
    winners = _winner_call(x, kt2, x2, w2).reshape(B)
    table = jnp.pad(grossberg_weights.T, ((0, 0), (0, _DPAD - OUT)))
    output = _gather_call(table, winners.reshape(_NW, _NCHI, _CHI))[:, :OUT]
    return (output, winners)
